# baseline (speedup 1.0000x reference)
import functools

import jax
import jax.numpy as jnp
import numpy as np
from jax import lax
from jax.experimental import pallas as pl
from jax.experimental.pallas import tpu as pltpu
from jax.experimental.pallas import tpu_sc as plsc

_TABLE = 40960
_H = 256
_B = 16384
_L = 20

_NC = 2
_NS = 16
_NW = _NC * _NS
_NSEG = 2 * _B
_SEG_PER_W = _NSEG // _NW
_CHUNK = 4
_NCHUNK = _SEG_PER_W // _CHUNK
_LANES = 16


def _sc_gather_sum(table, idx_flat):
    mesh = plsc.VectorSubcoreMesh(core_axis_name="c", subcore_axis_name="s")

    nbuf = 8
    ngrp = _NCHUNK // nbuf

    @functools.partial(
        pl.kernel,
        out_type=jax.ShapeDtypeStruct((_NSEG, _H), jnp.float32),
        mesh=mesh,
        scratch_types=[
            pltpu.VMEM((_SEG_PER_W * _L,), jnp.int32),
            [pltpu.VMEM((_CHUNK * _L, _H // 2), jnp.int32)
             for _ in range(nbuf)],
            [pltpu.VMEM((_CHUNK, _H), jnp.float32) for _ in range(nbuf)],
            [pltpu.SemaphoreType.DMA for _ in range(nbuf)],
            [pltpu.SemaphoreType.DMA for _ in range(nbuf)],
        ],
    )
    def k(table_hbm, idx_hbm, out_hbm, idx_v, rows, outs, sems, osems):
        wid = lax.axis_index("s") * _NC + lax.axis_index("c")
        idx_base = wid * (_SEG_PER_W * _L)
        seg_base = wid * _SEG_PER_W
        pltpu.sync_copy(idx_hbm.at[pl.ds(idx_base, _SEG_PER_W * _L)], idx_v)

        def issue_gather(g, p):
            hl = _CHUNK * _L // 2
            base = g * (_CHUNK * _L)
            pltpu.async_copy(table_hbm.at[idx_v.at[pl.ds(base, hl)]],
                             rows[p].at[pl.ds(0, hl)], sems[p])
            pltpu.async_copy(table_hbm.at[idx_v.at[pl.ds(base + hl, hl)]],
                             rows[p].at[pl.ds(hl, hl)], sems[p])

        def wait_gather(p):
            pltpu.make_async_copy(table_hbm.at[pl.ds(0, _CHUNK * _L)],
                                  rows[p], sems[p]).wait()

        def accumulate(p):
            mask_hi = jnp.full((_LANES,), -65536, jnp.int32)

            def unpack2(row, sl):
                x = rows[p][row, sl]
                lo = lax.bitcast_convert_type(lax.shift_left(x, 16),
                                              jnp.float32)
                hi = lax.bitcast_convert_type(lax.bitwise_and(x, mask_hi),
                                              jnp.float32)
                return lo, hi

            def seg_body(c, _):
                base = c * _L
                for hh in range(_H // 32):
                    sl = pl.ds(hh * _LANES, _LANES)
                    lo_parts, hi_parts = [], []
                    for q in range(4):
                        a_lo, a_hi = unpack2(base + 5 * q, sl)
                        for r in range(1, 5):
                            b_lo, b_hi = unpack2(base + 5 * q + r, sl)
                            a_lo = a_lo + b_lo
                            a_hi = a_hi + b_hi
                        lo_parts.append(a_lo)
                        hi_parts.append(a_hi)
                    lo = (lo_parts[0] + lo_parts[1]) + (lo_parts[2]
                                                        + lo_parts[3])
                    hi = (hi_parts[0] + hi_parts[1]) + (hi_parts[2]
                                                        + hi_parts[3])
                    outs[p][c, pl.ds(hh * 32, _LANES)] = (
                        jnp.maximum(lo, 0.0))
                    outs[p][c, pl.ds(hh * 32 + _LANES, _LANES)] = (
                        jnp.maximum(hi, 0.0))
                return ()

            lax.fori_loop(0, _CHUNK, seg_body, (), unroll=False)

        def out_slice(g):
            return out_hbm.at[pl.ds(seg_base + g * _CHUNK, _CHUNK)]

        for p in range(nbuf):
            issue_gather(p, p)

        def body(j, _):
            for p in range(nbuf):
                g = j * nbuf + p
                wait_gather(p)

                @pl.when(j > 0)
                def _():
                    pltpu.make_async_copy(outs[p], out_slice(0),
                                          osems[p]).wait()

                accumulate(p)

                @pl.when(j < ngrp - 1)
                def _():
                    issue_gather(g + nbuf, p)

                pltpu.async_copy(outs[p], out_slice(g), osems[p])
            return ()

        lax.fori_loop(0, ngrp, body, (), unroll=False)
        for p in range(nbuf):
            pltpu.make_async_copy(outs[p], out_slice(0), osems[p]).wait()

    return k(table, idx_flat)


def _mlp_body(h_ref, w2_ref, b2_ref, w3_ref, b3_ref, w4_ref, b4_ref, out_ref):
    h = h_ref[...]
    z = jnp.maximum(
        jnp.dot(h, w2_ref[...], preferred_element_type=jnp.float32)
        + b2_ref[...], 0.0)
    z = jnp.maximum(
        jnp.dot(z, w3_ref[...], preferred_element_type=jnp.float32)
        + b3_ref[...], 0.0)
    out_ref[...] = jnp.sum(z * w4_ref[...], axis=1) + b4_ref[0, 0]


def _mlp(h, w2t, fc2_b, fc3_w, fc3_b, fc4_w, fc4_b):
    blk = 2048
    grid = (_B // blk,)
    full = lambda *s: pl.BlockSpec(s, lambda i: (0,) * len(s))
    return pl.pallas_call(
        _mlp_body,
        grid=grid,
        in_specs=[
            pl.BlockSpec((blk, 2 * _H), lambda i: (i, 0)),
            full(2 * _H, 32), full(1, 32),
            full(32, 32), full(1, 32),
            full(1, 32), full(1, 1),
        ],
        out_specs=pl.BlockSpec((blk,), lambda i: (i,)),
        out_shape=jax.ShapeDtypeStruct((_B,), jnp.float32),
    )(h, w2t, fc2_b.reshape(1, 32), fc3_w.T, fc3_b.reshape(1, 32),
      fc4_w.reshape(1, 32), fc4_b.reshape(1, 1))


def _permute_w2t(w2t):
    return w2t.reshape(2, 8, 16, 2, 32).transpose(0, 1, 3, 2, 4).reshape(
        2 * _H, 32)


def kernel(idx0_batch, idx1_batch, w1, fc2_w, fc2_b, fc3_w, fc3_b,
           fc4_w, fc4_b):
    tb = w1.reshape(2 * _TABLE, _H).astype(jnp.bfloat16)
    table_i32 = lax.bitcast_convert_type(
        tb.reshape(2 * _TABLE, _H // 2, 2), jnp.int32)
    idx_all = jnp.stack([idx0_batch, idx1_batch + _TABLE], axis=1)
    idx_flat = idx_all.reshape(-1)
    h = _sc_gather_sum(table_i32, idx_flat)
    h2 = h.reshape(_B, 2 * _H)
    w2t = _permute_w2t(fc2_w.T)
    return _mlp(h2, w2t, fc2_b, fc3_w, fc3_b, fc4_w, fc4_b)


# speedup vs baseline: 1.7654x; 1.7654x over previous
import functools

import jax
import jax.numpy as jnp
import numpy as np
from jax import lax
from jax.experimental import pallas as pl
from jax.experimental.pallas import tpu as pltpu
from jax.experimental.pallas import tpu_sc as plsc

_TABLE = 40960
_H = 256
_B = 16384
_L = 20

_NC = 2
_NS = 16
_NW = _NC * _NS
_NSEG = 2 * _B
_SEG_PER_W = _NSEG // _NW
_CHUNK = 4
_NCHUNK = _SEG_PER_W // _CHUNK
_LANES = 16


def _sc_gather_sum(table, idx_flat):
    mesh = plsc.VectorSubcoreMesh(core_axis_name="c", subcore_axis_name="s")

    nbuf = 8
    ngrp = _NCHUNK // nbuf

    @functools.partial(
        pl.kernel,
        out_type=jax.ShapeDtypeStruct((_NSEG, _H), jnp.float32),
        mesh=mesh,
        scratch_types=[
            pltpu.VMEM((_SEG_PER_W * _L,), jnp.int32),
            [pltpu.VMEM((_CHUNK * _L, _H // 2), jnp.int32)
             for _ in range(nbuf)],
            [pltpu.VMEM((_CHUNK, _H), jnp.float32) for _ in range(nbuf)],
            [pltpu.SemaphoreType.DMA for _ in range(nbuf)],
            [pltpu.SemaphoreType.DMA for _ in range(nbuf)],
        ],
    )
    def k(table_hbm, idx_hbm, out_hbm, idx_v, rows, outs, sems, osems):
        wid = lax.axis_index("s") * _NC + lax.axis_index("c")
        idx_base = wid * (_SEG_PER_W * _L)
        seg_base = wid * _SEG_PER_W
        pltpu.sync_copy(idx_hbm.at[pl.ds(idx_base, _SEG_PER_W * _L)], idx_v)

        def issue_gather(g, p):
            hl = _CHUNK * _L // 2
            base = g * (_CHUNK * _L)
            pltpu.async_copy(table_hbm.at[idx_v.at[pl.ds(base, hl)]],
                             rows[p].at[pl.ds(0, hl)], sems[p])
            pltpu.async_copy(table_hbm.at[idx_v.at[pl.ds(base + hl, hl)]],
                             rows[p].at[pl.ds(hl, hl)], sems[p])

        def wait_gather(p):
            pltpu.make_async_copy(table_hbm.at[pl.ds(0, _CHUNK * _L)],
                                  rows[p], sems[p]).wait()

        def accumulate(p):
            mask_hi = jnp.full((_LANES,), -65536, jnp.int32)

            def unpack2(row, sl):
                x = rows[p][row, sl]
                lo = lax.bitcast_convert_type(lax.shift_left(x, 16),
                                              jnp.float32)
                hi = lax.bitcast_convert_type(lax.bitwise_and(x, mask_hi),
                                              jnp.float32)
                return lo, hi

            def seg_body(c, _):
                base = c * _L
                for hh in range(_H // 32):
                    sl = pl.ds(hh * _LANES, _LANES)
                    lo_parts, hi_parts = [], []
                    for q in range(4):
                        a_lo, a_hi = unpack2(base + 5 * q, sl)
                        for r in range(1, 5):
                            b_lo, b_hi = unpack2(base + 5 * q + r, sl)
                            a_lo = a_lo + b_lo
                            a_hi = a_hi + b_hi
                        lo_parts.append(a_lo)
                        hi_parts.append(a_hi)
                    lo = (lo_parts[0] + lo_parts[1]) + (lo_parts[2]
                                                        + lo_parts[3])
                    hi = (hi_parts[0] + hi_parts[1]) + (hi_parts[2]
                                                        + hi_parts[3])
                    outs[p][c, pl.ds(hh * _LANES, _LANES)] = (
                        jnp.maximum(lo, 0.0))
                    outs[p][c, pl.ds(_H // 2 + hh * _LANES, _LANES)] = (
                        jnp.maximum(hi, 0.0))
                return ()

            lax.fori_loop(0, _CHUNK, seg_body, (), unroll=False)

        def out_slice(g):
            return out_hbm.at[pl.ds(seg_base + g * _CHUNK, _CHUNK)]

        for p in range(nbuf):
            issue_gather(p, p)

        def body(j, _):
            for p in range(nbuf):
                g = j * nbuf + p
                wait_gather(p)

                @pl.when(j > 0)
                def _():
                    pltpu.make_async_copy(outs[p], out_slice(0),
                                          osems[p]).wait()

                accumulate(p)

                @pl.when(j < ngrp - 1)
                def _():
                    issue_gather(g + nbuf, p)

                pltpu.async_copy(outs[p], out_slice(g), osems[p])
            return ()

        lax.fori_loop(0, ngrp, body, (), unroll=False)
        for p in range(nbuf):
            pltpu.make_async_copy(outs[p], out_slice(0), osems[p]).wait()

    return k(table, idx_flat)


def _mlp_body(h_ref, w2_ref, b2_ref, w3_ref, b3_ref, w4_ref, b4_ref, out_ref):
    h = h_ref[...]
    z = jnp.maximum(
        jnp.dot(h, w2_ref[...], preferred_element_type=jnp.float32)
        + b2_ref[...], 0.0)
    z = jnp.maximum(
        jnp.dot(z, w3_ref[...], preferred_element_type=jnp.float32)
        + b3_ref[...], 0.0)
    out_ref[...] = jnp.sum(z * w4_ref[...], axis=1) + b4_ref[0, 0]


def _mlp(h, w2t, fc2_b, fc3_w, fc3_b, fc4_w, fc4_b):
    blk = 2048
    grid = (_B // blk,)
    full = lambda *s: pl.BlockSpec(s, lambda i: (0,) * len(s))
    return pl.pallas_call(
        _mlp_body,
        grid=grid,
        in_specs=[
            pl.BlockSpec((blk, 2 * _H), lambda i: (i, 0)),
            full(2 * _H, 32), full(1, 32),
            full(32, 32), full(1, 32),
            full(1, 32), full(1, 1),
        ],
        out_specs=pl.BlockSpec((blk,), lambda i: (i,)),
        out_shape=jax.ShapeDtypeStruct((_B,), jnp.float32),
    )(h, w2t, fc2_b.reshape(1, 32), fc3_w.T, fc3_b.reshape(1, 32),
      fc4_w.reshape(1, 32), fc4_b.reshape(1, 1))


def kernel(idx0_batch, idx1_batch, w1, fc2_w, fc2_b, fc3_w, fc3_b,
           fc4_w, fc4_b):
    tb = w1.reshape(2 * _TABLE, _H).astype(jnp.bfloat16)
    u16 = lax.bitcast_convert_type(tb, jnp.uint16)
    lo = u16[:, :_H // 2].astype(jnp.uint32)
    hi = u16[:, _H // 2:].astype(jnp.uint32)
    table_i32 = lax.bitcast_convert_type(
        lo | (hi << jnp.uint32(16)), jnp.int32)
    idx_all = jnp.stack([idx0_batch, idx1_batch + _TABLE], axis=1)
    idx_flat = idx_all.reshape(-1)
    h = _sc_gather_sum(table_i32, idx_flat)
    h2 = h.reshape(_B, 2 * _H)
    return _mlp(h2, fc2_w.T, fc2_b, fc3_w, fc3_b, fc4_w, fc4_b)


# speedup vs baseline: 1.8619x; 1.0547x over previous
import functools

import jax
import jax.numpy as jnp
import numpy as np
from jax import lax
from jax.experimental import pallas as pl
from jax.experimental.pallas import tpu as pltpu
from jax.experimental.pallas import tpu_sc as plsc

_TABLE = 40960
_H = 256
_B = 16384
_L = 20

_NC = 2
_NS = 16
_NW = _NC * _NS
_NSEG = 2 * _B
_SEG_PER_W = _NSEG // _NW
_CHUNK = 4
_NCHUNK = _SEG_PER_W // _CHUNK
_LANES = 16


def _sc_gather_sum(table, idx_flat):
    mesh = plsc.VectorSubcoreMesh(core_axis_name="c", subcore_axis_name="s")

    nbuf = 8
    ngrp = _NCHUNK // nbuf

    @functools.partial(
        pl.kernel,
        out_type=jax.ShapeDtypeStruct((_NSEG, _H), jnp.float32),
        mesh=mesh,
        scratch_types=[
            pltpu.VMEM((_SEG_PER_W * _L,), jnp.int32),
            [pltpu.VMEM((_CHUNK * _L, _H // 2), jnp.int32)
             for _ in range(nbuf)],
            [pltpu.VMEM((_CHUNK, _H), jnp.float32) for _ in range(nbuf)],
            [pltpu.SemaphoreType.DMA for _ in range(nbuf)],
            [pltpu.SemaphoreType.DMA for _ in range(nbuf)],
        ],
    )
    def k(table_hbm, idx_hbm, out_hbm, idx_v, rows, outs, sems, osems):
        wid = lax.axis_index("s") * _NC + lax.axis_index("c")
        idx_base = wid * (_SEG_PER_W * _L)
        seg_base = wid * _SEG_PER_W
        pltpu.sync_copy(idx_hbm.at[pl.ds(idx_base, _SEG_PER_W * _L)], idx_v)

        def issue_gather(g, p):
            hl = _CHUNK * _L // 2
            base = g * (_CHUNK * _L)
            pltpu.async_copy(table_hbm.at[idx_v.at[pl.ds(base, hl)]],
                             rows[p].at[pl.ds(0, hl)], sems[p])
            pltpu.async_copy(table_hbm.at[idx_v.at[pl.ds(base + hl, hl)]],
                             rows[p].at[pl.ds(hl, hl)], sems[p])

        def wait_gather(p):
            pltpu.make_async_copy(table_hbm.at[pl.ds(0, _CHUNK * _L)],
                                  rows[p], sems[p]).wait()

        def accumulate(p):
            mask_hi = jnp.full((_LANES,), -65536, jnp.int32)

            def unpack2(row, sl):
                x = rows[p][row, sl]
                lo = lax.bitcast_convert_type(lax.shift_left(x, 16),
                                              jnp.float32)
                hi = lax.bitcast_convert_type(lax.bitwise_and(x, mask_hi),
                                              jnp.float32)
                return lo, hi

            def seg_body(c, _):
                base = c * _L
                for hh in range(_H // 32):
                    sl = pl.ds(hh * _LANES, _LANES)
                    lo_parts, hi_parts = [], []
                    for q in range(4):
                        a_lo, a_hi = unpack2(base + 5 * q, sl)
                        for r in range(1, 5):
                            b_lo, b_hi = unpack2(base + 5 * q + r, sl)
                            a_lo = a_lo + b_lo
                            a_hi = a_hi + b_hi
                        lo_parts.append(a_lo)
                        hi_parts.append(a_hi)
                    lo = (lo_parts[0] + lo_parts[1]) + (lo_parts[2]
                                                        + lo_parts[3])
                    hi = (hi_parts[0] + hi_parts[1]) + (hi_parts[2]
                                                        + hi_parts[3])
                    outs[p][c, pl.ds(hh * _LANES, _LANES)] = (
                        jnp.maximum(lo, 0.0))
                    outs[p][c, pl.ds(_H // 2 + hh * _LANES, _LANES)] = (
                        jnp.maximum(hi, 0.0))
                return ()

            lax.fori_loop(0, _CHUNK, seg_body, (), unroll=False)

        def out_slice(g):
            return out_hbm.at[pl.ds(seg_base + g * _CHUNK, _CHUNK)]

        for p in range(nbuf):
            issue_gather(p, p)

        def body(j, _):
            for p in range(nbuf):
                g = j * nbuf + p
                wait_gather(p)

                @pl.when(j > 0)
                def _():
                    pltpu.make_async_copy(outs[p], out_slice(0),
                                          osems[p]).wait()

                accumulate(p)

                @pl.when(j < ngrp - 1)
                def _():
                    issue_gather(g + nbuf, p)

                pltpu.async_copy(outs[p], out_slice(g), osems[p])
            return ()

        lax.fori_loop(0, ngrp, body, (), unroll=False)
        for p in range(nbuf):
            pltpu.make_async_copy(outs[p], out_slice(0), osems[p]).wait()

    return k(table, idx_flat)


def _mlp_body(h_ref, w2_ref, b2_ref, w3_ref, b3_ref, w4_ref, b4_ref, out_ref):
    h = h_ref[...]
    z = jnp.maximum(
        jnp.dot(h, w2_ref[...], preferred_element_type=jnp.float32)
        + b2_ref[...], 0.0)
    z = jnp.maximum(
        jnp.dot(z, w3_ref[...], preferred_element_type=jnp.float32)
        + b3_ref[...], 0.0)
    out_ref[...] = jnp.sum(z * w4_ref[...], axis=1) + b4_ref[0, 0]


def _mlp(h, w2t, fc2_b, fc3_w, fc3_b, fc4_w, fc4_b):
    blk = 2048
    grid = (_B // blk,)
    full = lambda *s: pl.BlockSpec(s, lambda i: (0,) * len(s))
    return pl.pallas_call(
        _mlp_body,
        grid=grid,
        in_specs=[
            pl.BlockSpec((blk, 2 * _H), lambda i: (i, 0)),
            full(2 * _H, 32), full(1, 32),
            full(32, 32), full(1, 32),
            full(1, 32), full(1, 1),
        ],
        out_specs=pl.BlockSpec((blk,), lambda i: (i,)),
        out_shape=jax.ShapeDtypeStruct((_B,), jnp.float32),
    )(h, w2t, fc2_b.reshape(1, 32), fc3_w.T, fc3_b.reshape(1, 32),
      fc4_w.reshape(1, 32), fc4_b.reshape(1, 1))


def _pack_body(w_ref, o_ref):
    u = lax.bitcast_convert_type(w_ref[...], jnp.uint32)
    r = (u + jnp.uint32(0x7FFF)
         + ((u >> jnp.uint32(16)) & jnp.uint32(1))) >> jnp.uint32(16)
    packed = r[:, :_H // 2] | (r[:, _H // 2:] << jnp.uint32(16))
    o_ref[...] = lax.bitcast_convert_type(packed, jnp.int32)


def _pack_table(w):
    blk = 4096
    return pl.pallas_call(
        _pack_body,
        grid=(2 * _TABLE // blk,),
        in_specs=[pl.BlockSpec((blk, _H), lambda i: (i, 0))],
        out_specs=pl.BlockSpec((blk, _H // 2), lambda i: (i, 0)),
        out_shape=jax.ShapeDtypeStruct((2 * _TABLE, _H // 2), jnp.int32),
    )(w)


def kernel(idx0_batch, idx1_batch, w1, fc2_w, fc2_b, fc3_w, fc3_b,
           fc4_w, fc4_b):
    table_i32 = _pack_table(w1.reshape(2 * _TABLE, _H))
    idx_all = jnp.stack([idx0_batch, idx1_batch + _TABLE], axis=1)
    idx_flat = idx_all.reshape(-1)
    h = _sc_gather_sum(table_i32, idx_flat)
    h2 = h.reshape(_B, 2 * _H)
    return _mlp(h2, fc2_w.T, fc2_b, fc3_w, fc3_b, fc4_w, fc4_b)
